# baseline (speedup 1.0000x reference)
import jax
import jax.numpy as jnp
from jax import lax
from jax.experimental import pallas as pl
from jax.experimental.pallas import tpu as pltpu
from jax.experimental.pallas import tpu_sc as plsc

EMB = 16
MARGIN = 0.1
NC = 2
NS = 16
NW = NC * NS
TERMS = 32768
TPW = TERMS // NW
CHUNK = 128
NCH = TPW // CHUNK
BLOCKS = TPW // 16


def _sc_body(idx_hbm, ent_hbm, rel_hbm, out_hbm,
             idx_v, hp, rp, tp, hn, rn, tn, sqp_v, sqn_v, sem):
    wid = lax.axis_index("s") * NC + lax.axis_index("c")
    pltpu.sync_copy(idx_hbm.at[wid], idx_v)

    bufs = (hp, rp, tp, hn, rn, tn)
    tables = (ent_hbm, rel_hbm, ent_hbm, ent_hbm, rel_hbm, ent_hbm)
    copies = []
    for j in range(6):
        for c in range(NCH):
            copies.append(pltpu.async_copy(
                tables[j].at[idx_v.at[j, c]],
                bufs[j].at[pl.ds(c * CHUNK, CHUNK)], sem))
    for cp in copies:
        cp.wait()

    iot = lax.iota(jnp.int32, 16)
    perms = [iot ^ s for s in (8, 4, 2, 1)]
    masks = [(iot & s) == 0 for s in (8, 4, 2, 1)]

    dnums = lax.GatherDimensionNumbers(
        offset_dims=(), collapsed_slice_dims=(0,), start_index_map=(0,))

    def _perm(v, pidx):
        return lax.gather(v, pidx[:, None], dnums, (1,),
                          mode=lax.GatherScatterMode.PROMISE_IN_BOUNDS)

    def _rowsums(h, r, t, base):
        es = []
        for j in range(16):
            d = h[base + j, :] + r[base + j, :] - t[base + j, :]
            es.append(d * d)
        for pidx, msk in zip(perms, masks):
            half = len(es) // 2
            es = [jnp.where(msk,
                            es[i] + _perm(es[i], pidx),
                            es[i + half] + _perm(es[i + half], pidx))
                  for i in range(half)]
        return es[0]

    def block(b, carry):
        base = b * 16
        sqp_v[pl.ds(base, 16)] = _rowsums(hp, rp, tp, base)
        sqn_v[pl.ds(base, 16)] = _rowsums(hn, rn, tn, base)
        return carry

    lax.fori_loop(0, BLOCKS, block, jnp.int32(0))
    pltpu.sync_copy(sqp_v, out_hbm.at[0, pl.ds(wid * TPW, TPW)])
    pltpu.sync_copy(sqn_v, out_hbm.at[1, pl.ds(wid * TPW, TPW)])


def _loss_body(x_ref, o_ref):
    sp = x_ref[0, :]
    sn = x_ref[1, :]
    loss = jnp.maximum(MARGIN + jnp.sqrt(sp) - jnp.sqrt(sn), 0.0)
    o_ref[...] = jnp.sum(loss).reshape(1, 1)


def kernel(lhs_pos, rhs_pos, lhs_neg, rhs_neg, ent_emb, rel_emb):
    pos = jnp.concatenate([lhs_pos, rhs_pos], axis=0).astype(jnp.int32)
    neg = jnp.concatenate([lhs_neg, rhs_neg], axis=0).astype(jnp.int32)
    allidx = jnp.stack([pos[:, 0], pos[:, 1], pos[:, 2],
                        neg[:, 0], neg[:, 1], neg[:, 2]])
    idx = (allidx.reshape(6, NW, TPW).transpose(1, 0, 2)
           .reshape(NW, 6, NCH, CHUNK))

    mesh = plsc.VectorSubcoreMesh(core_axis_name="c", subcore_axis_name="s")
    sc = pl.kernel(
        _sc_body,
        out_type=jax.ShapeDtypeStruct((2, TERMS), jnp.float32),
        mesh=mesh,
        scratch_types=[
            pltpu.VMEM((6, NCH, CHUNK), jnp.int32),
            pltpu.VMEM((TPW, EMB), jnp.float32),
            pltpu.VMEM((TPW, EMB), jnp.float32),
            pltpu.VMEM((TPW, EMB), jnp.float32),
            pltpu.VMEM((TPW, EMB), jnp.float32),
            pltpu.VMEM((TPW, EMB), jnp.float32),
            pltpu.VMEM((TPW, EMB), jnp.float32),
            pltpu.VMEM((TPW,), jnp.float32),
            pltpu.VMEM((TPW,), jnp.float32),
            pltpu.SemaphoreType.DMA,
        ],
        compiler_params=pltpu.CompilerParams(use_tc_tiling_on_sc=False),
    )
    sq = sc(idx, ent_emb, rel_emb)

    loss = pl.pallas_call(
        _loss_body,
        out_shape=jax.ShapeDtypeStruct((1, 1), jnp.float32),
    )(sq)
    return loss[0, 0]
